# Initial kernel scaffold; baseline (speedup 1.0000x reference)
import jax
import jax.numpy as jnp
from jax.experimental import pallas as pl


def kernel(points, edge_index, kernel_points, W):
    raise NotImplementedError("write your pallas kernel here")


# speedup vs baseline: 5.4323x; 5.4323x over previous
import functools

import jax
import jax.numpy as jnp
from jax import lax
from jax.experimental import pallas as pl
from jax.experimental.pallas import tpu as pltpu
from jax.experimental.pallas import tpu_sc as plsc

NC = 2
NS = 16
NW = NC * NS
L = 16

SIGMA = 0.1
K = 15
D_OUT = 64

B1 = 2048
B3 = 2048
CHUNK = 128


def _iota16():
  return lax.iota(jnp.int32, L)


def _sc_gather_body(n_nodes, px_hbm, py_hbm, pz_hbm, src_hbm, dst_hbm, g_hbm,
                    px, py, pz, stage, idx_s, idx_d,
                    c0, c1, c2, c3, c4, c5):
  c = lax.axis_index("c")
  t = lax.axis_index("s")
  wid = t * NC + c
  e_pad = g_hbm.shape[1]
  per_w = e_pad // NW
  nbatch = per_w // B1
  n_per_t = n_nodes // NS
  cols = (c0, c1, c2, c3, c4, c5)
  tabs = (px, py, pz)

  for cc, p_hbm in enumerate((px_hbm, py_hbm, pz_hbm)):
    pltpu.sync_copy(p_hbm.at[pl.ds(t * n_per_t, n_per_t)], stage)
    pltpu.sync_copy(stage, tabs[cc].at[pl.ds(t * n_per_t, n_per_t)])
  plsc.subcore_barrier()

  def batch(b, _):
    base = wid * per_w + b * B1
    pltpu.sync_copy(src_hbm.at[pl.ds(base, B1)], idx_s)
    pltpu.sync_copy(dst_hbm.at[pl.ds(base, B1)], idx_d)
    for cc in range(3):
      pltpu.sync_copy(tabs[cc].at[idx_s], cols[cc])
      pltpu.sync_copy(tabs[cc].at[idx_d], cols[3 + cc])
    for r in range(6):
      pltpu.sync_copy(cols[r], g_hbm.at[r, pl.ds(base, B1)])
    return _

  lax.fori_loop(0, nbatch, batch, 0)


def _sc_gather(posT, srcp, dstp, n_nodes):
  e_pad = srcp.shape[0]
  px, py, pz = posT[0], posT[1], posT[2]
  body = functools.partial(_sc_gather_body, n_nodes)
  fn = pl.kernel(
      body,
      out_type=jax.ShapeDtypeStruct((6, e_pad), jnp.float32),
      mesh=plsc.VectorSubcoreMesh(core_axis_name="c", subcore_axis_name="s"),
      scratch_types=[
          pltpu.MemorySpace.VMEM_SHARED((n_nodes,), jnp.float32),
          pltpu.MemorySpace.VMEM_SHARED((n_nodes,), jnp.float32),
          pltpu.MemorySpace.VMEM_SHARED((n_nodes,), jnp.float32),
          pltpu.VMEM((n_nodes // NS,), jnp.float32),
          pltpu.VMEM((B1,), jnp.int32),
          pltpu.VMEM((B1,), jnp.int32),
      ] + [pltpu.VMEM((B1,), jnp.float32)] * 6,
  )
  return fn(px, py, pz, srcp, dstp)


def _tc_compute_body(e_real, br, kp_ref, g_ref, v_ref):
  pid = pl.program_id(0)
  fx, fy, fz = g_ref[0], g_ref[1], g_ref[2]
  rx = fx - g_ref[3]
  ry = fy - g_ref[4]
  rz = fz - g_ref[5]
  rows = lax.broadcasted_iota(jnp.int32, (br, 128), 0)
  lanes = lax.broadcasted_iota(jnp.int32, (br, 128), 1)
  gid = (pid * br + rows) * 128 + lanes
  valid = gid < e_real
  one = jnp.where(valid, 1.0, 0.0).astype(jnp.float32)
  fxm = fx * one
  fym = fy * one
  fzm = fz * one
  for k in range(K):
    dx = rx - kp_ref[k, 0]
    dy = ry - kp_ref[k, 1]
    dz = rz - kp_ref[k, 2]
    d2 = dx * dx + dy * dy + dz * dz + 1e-12
    dist = jnp.sqrt(d2)
    infl = jnp.maximum(0.0, 1.0 - dist * (1.0 / SIGMA))
    v_ref[3 * k + 0] = infl * fxm
    v_ref[3 * k + 1] = infl * fym
    v_ref[3 * k + 2] = infl * fzm
  v_ref[45] = one
  zero = jnp.zeros((br, 128), jnp.float32)
  v_ref[46] = zero
  v_ref[47] = zero


def _tc_compute(g3, kp, e_real):
  rows_tot = g3.shape[1]
  br = 256
  grid = rows_tot // br
  body = functools.partial(_tc_compute_body, e_real, br)
  return pl.pallas_call(
      body,
      grid=(grid,),
      in_specs=[
          pl.BlockSpec(memory_space=pltpu.SMEM),
          pl.BlockSpec((6, br, 128), lambda i: (0, i, 0)),
      ],
      out_specs=pl.BlockSpec((48, br, 128), lambda i: (0, i, 0)),
      out_shape=jax.ShapeDtypeStruct((48, rows_tot, 128), jnp.float32),
  )(kp, g3)


def _sc_scatter_body(n_nodes, e_pad, dst2_hbm, v_hbm, acc_hbm,
                     acc_sh, idxb, zbuf, wbuf, sem,
                     b0, b1, b2, b3, b4, b5, b6, b7):
  c = lax.axis_index("c")
  t = lax.axis_index("s")
  per_t = e_pad // NS
  nbatch = per_t // B3
  n_per_t = n_nodes // NS
  cbufs = (b0, b1, b2, b3, b4, b5, b6, b7)

  def zgrp(g, _):
    zbuf[pl.ds(g * L, L)] = jnp.zeros((L,), jnp.float32)
    return _
  lax.fori_loop(0, n_per_t // L, zgrp, 0)

  for sl in range(3):
    sg = 3 * c + sl
    for cl in range(8):
      off = pl.multiple_of(cl * n_nodes + t * n_per_t, n_per_t)
      pltpu.sync_copy(zbuf, acc_sh.at[pl.ds(off, n_per_t)])
    plsc.subcore_barrier()

    def batch(b, _):
      ebase = pl.multiple_of(t * per_t + b * B3, B3)
      rbase = pl.multiple_of(ebase // 128, B3 // 128)
      pltpu.sync_copy(dst2_hbm.at[pl.ds(rbase, B3 // 128)], idxb)
      ldescs = [
          pltpu.async_copy(v_hbm.at[sg * 8 + cl, pl.ds(ebase, B3)],
                           cbufs[cl], sem)
          for cl in range(8)
      ]
      for d in ldescs:
        d.wait()
      sdescs = []
      for cl in range(8):
        col = acc_sh.at[pl.ds(pl.multiple_of(cl * n_nodes, 8), n_nodes)]
        for j in range(B3 // CHUNK):
          sdescs.append(
              pltpu.async_copy(cbufs[cl].at[pl.ds(j * CHUNK, CHUNK)],
                               col.at[idxb.at[j]], sem, add=True))
      for d in sdescs:
        d.wait()
      return _

    lax.fori_loop(0, nbatch, batch, 0)
    plsc.subcore_barrier()
    for cl in range(8):
      off = pl.multiple_of(cl * n_nodes + t * n_per_t, n_per_t)
      nbase = pl.multiple_of(t * n_per_t, n_per_t)
      pltpu.sync_copy(acc_sh.at[pl.ds(off, n_per_t)], wbuf)
      hoff = pl.multiple_of((sg * 8 + cl) * n_nodes + nbase, n_per_t)
      pltpu.sync_copy(wbuf, acc_hbm.at[pl.ds(hoff, n_per_t)])
    plsc.subcore_barrier()


def _sc_scatter(dst2, v, n_nodes):
  e_pad = v.shape[1]
  body = functools.partial(_sc_scatter_body, n_nodes, e_pad)
  fn = pl.kernel(
      body,
      out_type=jax.ShapeDtypeStruct((48 * n_nodes,), jnp.float32),
      mesh=plsc.VectorSubcoreMesh(core_axis_name="c", subcore_axis_name="s"),
      scratch_types=[
          pltpu.MemorySpace.VMEM_SHARED((8 * n_nodes,), jnp.float32),
          pltpu.VMEM((B3 // 128, 128), jnp.int32),
          pltpu.VMEM((n_nodes // NS,), jnp.float32),
          pltpu.VMEM((n_nodes // NS,), jnp.float32),
          pltpu.SemaphoreType.DMA,
      ] + [pltpu.VMEM((B3,), jnp.float32)] * 8,
  )
  return fn(dst2, v)


def _tc_out_body(bn, acc_ref, wf_ref, out_ref):
  acc2 = acc_ref[...].reshape(48, bn)
  mm = jax.lax.dot_general(acc2, wf_ref[...], (((0,), (0,)), ((), ())),
                           preferred_element_type=jnp.float32)
  deg = jax.lax.broadcast_in_dim(acc2[45], (bn, D_OUT), (0,))
  y = mm / jnp.maximum(deg, 1.0)
  out_ref[...] = jnp.where(y >= 0.0, y, 0.1 * y)


def _tc_out(acc3, wf):
  n_nodes = acc3.shape[2]
  bn = 5888
  grid = n_nodes // bn
  body = functools.partial(_tc_out_body, bn)
  return pl.pallas_call(
      body,
      grid=(grid,),
      in_specs=[
          pl.BlockSpec((6, 8, bn), lambda i: (0, 0, i)),
          pl.BlockSpec((48, D_OUT), lambda i: (0, 0)),
      ],
      out_specs=pl.BlockSpec((bn, D_OUT), lambda i: (i, 0)),
      out_shape=jax.ShapeDtypeStruct((n_nodes, D_OUT), jnp.float32),
  )(acc3, wf)


def kernel(points, edge_index, kernel_points, W):
  n_nodes = points.shape[0]
  e_real = edge_index.shape[1]
  step = NW * B1
  e_pad = ((e_real + step - 1) // step) * step

  n_pad = ((n_nodes + 127) // 128) * 128

  pos = points[:, 1:4].astype(jnp.float32)
  posT = jnp.pad(pos.T, ((0, 0), (0, n_pad - n_nodes)))
  src = edge_index[0].astype(jnp.int32)
  dst = edge_index[1].astype(jnp.int32)
  srcp = jnp.pad(src, (0, e_pad - e_real))
  dstp = jnp.pad(dst, (0, e_pad - e_real))

  kp = kernel_points.astype(jnp.float32)

  g = _sc_gather(posT, srcp, dstp, n_pad)
  g3 = g.reshape(6, e_pad // 128, 128)
  v3 = _tc_compute(g3, kp, e_real)
  v = v3.reshape(48, e_pad)

  dst2 = dstp.reshape(e_pad // 128, 128)
  acc = _sc_scatter(dst2, v, n_pad).reshape(6, 8, n_pad)

  wf = W.reshape(K * 3, D_OUT).astype(jnp.float32)
  wf = jnp.concatenate([wf, jnp.zeros((3, D_OUT), jnp.float32)], axis=0)
  return _tc_out(acc, wf)[:n_nodes]


# speedup vs baseline: 6.0147x; 1.1072x over previous
import functools

import jax
import jax.numpy as jnp
from jax import lax
from jax.experimental import pallas as pl
from jax.experimental.pallas import tpu as pltpu
from jax.experimental.pallas import tpu_sc as plsc

NC = 2
NS = 16
NW = NC * NS
L = 16

SIGMA = 0.1
K = 15
D_OUT = 64

B1 = 2048
B3 = 2048
CHUNK = 128


def _iota16():
  return lax.iota(jnp.int32, L)


def _sc_gather_body(n_nodes, px_hbm, py_hbm, pz_hbm, src_hbm, dst_hbm, g_hbm,
                    px, py, pz, stage, idx_s, idx_d,
                    c0, c1, c2, c3, c4, c5):
  c = lax.axis_index("c")
  t = lax.axis_index("s")
  wid = t * NC + c
  e_pad = g_hbm.shape[1]
  per_w = e_pad // NW
  nbatch = per_w // B1
  n_per_t = n_nodes // NS
  cols = (c0, c1, c2, c3, c4, c5)
  tabs = (px, py, pz)

  for cc, p_hbm in enumerate((px_hbm, py_hbm, pz_hbm)):
    pltpu.sync_copy(p_hbm.at[pl.ds(t * n_per_t, n_per_t)], stage)
    pltpu.sync_copy(stage, tabs[cc].at[pl.ds(t * n_per_t, n_per_t)])
  plsc.subcore_barrier()

  def batch(b, _):
    base = wid * per_w + b * B1
    pltpu.sync_copy(src_hbm.at[pl.ds(base, B1)], idx_s)
    pltpu.sync_copy(dst_hbm.at[pl.ds(base, B1)], idx_d)
    for cc in range(3):
      pltpu.sync_copy(tabs[cc].at[idx_s], cols[cc])
      pltpu.sync_copy(tabs[cc].at[idx_d], cols[3 + cc])
    for r in range(6):
      pltpu.sync_copy(cols[r], g_hbm.at[r, pl.ds(base, B1)])
    return _

  lax.fori_loop(0, nbatch, batch, 0)


def _sc_gather(posT, srcp, dstp, n_nodes):
  e_pad = srcp.shape[0]
  px, py, pz = posT[0], posT[1], posT[2]
  body = functools.partial(_sc_gather_body, n_nodes)
  fn = pl.kernel(
      body,
      out_type=jax.ShapeDtypeStruct((6, e_pad), jnp.float32),
      mesh=plsc.VectorSubcoreMesh(core_axis_name="c", subcore_axis_name="s"),
      scratch_types=[
          pltpu.MemorySpace.VMEM_SHARED((n_nodes,), jnp.float32),
          pltpu.MemorySpace.VMEM_SHARED((n_nodes,), jnp.float32),
          pltpu.MemorySpace.VMEM_SHARED((n_nodes,), jnp.float32),
          pltpu.VMEM((n_nodes // NS,), jnp.float32),
          pltpu.VMEM((B1,), jnp.int32),
          pltpu.VMEM((B1,), jnp.int32),
      ] + [pltpu.VMEM((B1,), jnp.float32)] * 6,
  )
  return fn(px, py, pz, srcp, dstp)


def _tc_compute_body(e_real, br, kp_ref, g_ref, v_ref):
  pid = pl.program_id(0)
  fx, fy, fz = g_ref[0], g_ref[1], g_ref[2]
  rx = fx - g_ref[3]
  ry = fy - g_ref[4]
  rz = fz - g_ref[5]
  rows = lax.broadcasted_iota(jnp.int32, (br, 128), 0)
  lanes = lax.broadcasted_iota(jnp.int32, (br, 128), 1)
  gid = (pid * br + rows) * 128 + lanes
  valid = gid < e_real
  one = jnp.where(valid, 1.0, 0.0).astype(jnp.float32)
  fxm = fx * one
  fym = fy * one
  fzm = fz * one
  for k in range(K):
    dx = rx - kp_ref[k, 0]
    dy = ry - kp_ref[k, 1]
    dz = rz - kp_ref[k, 2]
    d2 = dx * dx + dy * dy + dz * dz + 1e-12
    dist = jnp.sqrt(d2)
    infl = jnp.maximum(0.0, 1.0 - dist * (1.0 / SIGMA))
    v_ref[3 * k + 0] = infl * fxm
    v_ref[3 * k + 1] = infl * fym
    v_ref[3 * k + 2] = infl * fzm
  v_ref[45] = one
  zero = jnp.zeros((br, 128), jnp.float32)
  v_ref[46] = zero
  v_ref[47] = zero


def _tc_compute(g3, kp, e_real):
  rows_tot = g3.shape[1]
  br = 256
  grid = rows_tot // br
  body = functools.partial(_tc_compute_body, e_real, br)
  return pl.pallas_call(
      body,
      grid=(grid,),
      in_specs=[
          pl.BlockSpec(memory_space=pltpu.SMEM),
          pl.BlockSpec((6, br, 128), lambda i: (0, i, 0)),
      ],
      out_specs=pl.BlockSpec((48, br, 128), lambda i: (0, i, 0)),
      out_shape=jax.ShapeDtypeStruct((48, rows_tot, 128), jnp.float32),
  )(kp, g3)


def _sc_scatter_body(n_nodes, e_pad, dst2_hbm, v4_hbm, acc_hbm,
                     acc_sh, zbuf, wbuf, semla, semlb, sems,
                     idxa, vbufa, idxb, vbufb):
  c = lax.axis_index("c")
  t = lax.axis_index("s")
  per_t = e_pad // NS
  nbatch = per_t // B3
  n_per_t = n_nodes // NS
  rows = B3 // 128

  def zgrp(g, _):
    zbuf[pl.ds(g * L, L)] = jnp.zeros((L,), jnp.float32)
    return _
  lax.fori_loop(0, n_per_t // L, zgrp, 0)

  def fire_loads(sg, b, idx, vbuf, sem):
    ebase = pl.multiple_of(t * per_t + b * B3, B3)
    rbase = pl.multiple_of(ebase // 128, rows)
    di = pltpu.async_copy(dst2_hbm.at[pl.ds(rbase, rows)], idx, sem)
    dv = pltpu.async_copy(
        v4_hbm.at[pl.ds(sg * 8, 8), pl.ds(rbase, rows), :], vbuf, sem)
    return di, dv

  def fire_scatters(idx, vbuf):
    ds = []
    for cl in range(8):
      col = acc_sh.at[pl.ds(pl.multiple_of(cl * n_nodes, 8), n_nodes)]
      for j in range(rows):
        ds.append(pltpu.async_copy(vbuf.at[cl, j], col.at[idx.at[j]],
                                   sems, add=True))
    return ds

  for sl in range(3):
    sg = 3 * c + sl
    for cl in range(8):
      off = pl.multiple_of(cl * n_nodes + t * n_per_t, n_per_t)
      pltpu.sync_copy(zbuf, acc_sh.at[pl.ds(off, n_per_t)])
    plsc.subcore_barrier()

    def batch2(i, _):
      b = 2 * i
      la = fire_loads(sg, b, idxa, vbufa, semla)
      lb = fire_loads(sg, b + 1, idxb, vbufb, semlb)
      for d in la:
        d.wait()
      sa = fire_scatters(idxa, vbufa)
      for d in lb:
        d.wait()
      sb = fire_scatters(idxb, vbufb)
      for d in sa + sb:
        d.wait()
      return _

    lax.fori_loop(0, nbatch // 2, batch2, 0)
    plsc.subcore_barrier()
    for cl in range(8):
      off = pl.multiple_of(cl * n_nodes + t * n_per_t, n_per_t)
      nbase = pl.multiple_of(t * n_per_t, n_per_t)
      pltpu.sync_copy(acc_sh.at[pl.ds(off, n_per_t)], wbuf)
      hoff = pl.multiple_of((sg * 8 + cl) * n_nodes + nbase, n_per_t)
      pltpu.sync_copy(wbuf, acc_hbm.at[pl.ds(hoff, n_per_t)])
    plsc.subcore_barrier()


def _sc_scatter(dst2, v4, n_nodes):
  e_pad = v4.shape[1] * 128
  body = functools.partial(_sc_scatter_body, n_nodes, e_pad)
  rows = B3 // 128
  fn = pl.kernel(
      body,
      out_type=jax.ShapeDtypeStruct((48 * n_nodes,), jnp.float32),
      mesh=plsc.VectorSubcoreMesh(core_axis_name="c", subcore_axis_name="s"),
      scratch_types=[
          pltpu.MemorySpace.VMEM_SHARED((8 * n_nodes,), jnp.float32),
          pltpu.VMEM((n_nodes // NS,), jnp.float32),
          pltpu.VMEM((n_nodes // NS,), jnp.float32),
          pltpu.SemaphoreType.DMA,
          pltpu.SemaphoreType.DMA,
          pltpu.SemaphoreType.DMA,
          pltpu.VMEM((rows, 128), jnp.int32),
          pltpu.VMEM((8, rows, 128), jnp.float32),
          pltpu.VMEM((rows, 128), jnp.int32),
          pltpu.VMEM((8, rows, 128), jnp.float32),
      ],
  )
  return fn(dst2, v4)


def _tc_out_body(bn, acc_ref, wf_ref, out_ref):
  acc2 = acc_ref[...].reshape(48, bn)
  mm = jax.lax.dot_general(acc2, wf_ref[...], (((0,), (0,)), ((), ())),
                           preferred_element_type=jnp.float32)
  deg = jax.lax.broadcast_in_dim(acc2[45], (bn, D_OUT), (0,))
  y = mm / jnp.maximum(deg, 1.0)
  out_ref[...] = jnp.where(y >= 0.0, y, 0.1 * y)


def _tc_out(acc3, wf):
  n_nodes = acc3.shape[2]
  bn = 5888
  grid = n_nodes // bn
  body = functools.partial(_tc_out_body, bn)
  return pl.pallas_call(
      body,
      grid=(grid,),
      in_specs=[
          pl.BlockSpec((6, 8, bn), lambda i: (0, 0, i)),
          pl.BlockSpec((48, D_OUT), lambda i: (0, 0)),
      ],
      out_specs=pl.BlockSpec((bn, D_OUT), lambda i: (i, 0)),
      out_shape=jax.ShapeDtypeStruct((n_nodes, D_OUT), jnp.float32),
  )(acc3, wf)


def kernel(points, edge_index, kernel_points, W):
  n_nodes = points.shape[0]
  e_real = edge_index.shape[1]
  step = NW * B1
  e_pad = ((e_real + step - 1) // step) * step

  n_pad = ((n_nodes + 127) // 128) * 128

  pos = points[:, 1:4].astype(jnp.float32)
  posT = jnp.pad(pos.T, ((0, 0), (0, n_pad - n_nodes)))
  src = edge_index[0].astype(jnp.int32)
  dst = edge_index[1].astype(jnp.int32)
  srcp = jnp.pad(src, (0, e_pad - e_real))
  dstp = jnp.pad(dst, (0, e_pad - e_real))

  kp = kernel_points.astype(jnp.float32)

  g = _sc_gather(posT, srcp, dstp, n_pad)
  g3 = g.reshape(6, e_pad // 128, 128)
  v3 = _tc_compute(g3, kp, e_real)

  dst2 = dstp.reshape(e_pad // 128, 128)
  acc = _sc_scatter(dst2, v3, n_pad).reshape(6, 8, n_pad)

  wf = W.reshape(K * 3, D_OUT).astype(jnp.float32)
  wf = jnp.concatenate([wf, jnp.zeros((3, D_OUT), jnp.float32)], axis=0)
  return _tc_out(acc, wf)[:n_nodes]


# speedup vs baseline: 6.0889x; 1.0123x over previous
import functools

import jax
import jax.numpy as jnp
from jax import lax
from jax.experimental import pallas as pl
from jax.experimental.pallas import tpu as pltpu
from jax.experimental.pallas import tpu_sc as plsc

NC = 2
NS = 16
NW = NC * NS
L = 16

SIGMA = 0.1
K = 15
D_OUT = 64

B1 = 2048
B3 = 2048
CHUNK = 128


def _iota16():
  return lax.iota(jnp.int32, L)


def _sc_gather_body(n_nodes, px_hbm, py_hbm, pz_hbm, src_hbm, dst_hbm, g_hbm,
                    px, py, pz, stage,
                    semia, semib, semga, semgb, semwa, semwb,
                    isa, ida, isb, idb, *colsets):
  c = lax.axis_index("c")
  t = lax.axis_index("s")
  wid = t * NC + c
  e_pad = g_hbm.shape[1]
  per_w = e_pad // NW
  nbatch = per_w // B1
  n_per_t = n_nodes // NS
  tabs = (px, py, pz)
  cols_a = colsets[0:6]
  cols_b = colsets[6:12]

  for cc, p_hbm in enumerate((px_hbm, py_hbm, pz_hbm)):
    pltpu.sync_copy(p_hbm.at[pl.ds(t * n_per_t, n_per_t)], stage)
    pltpu.sync_copy(stage, tabs[cc].at[pl.ds(t * n_per_t, n_per_t)])
  plsc.subcore_barrier()

  def fire_idx(b, i_s, i_d, sem):
    base = pl.multiple_of(wid * per_w + b * B1, B1)
    return (pltpu.async_copy(src_hbm.at[pl.ds(base, B1)], i_s, sem),
            pltpu.async_copy(dst_hbm.at[pl.ds(base, B1)], i_d, sem))

  def fire_gathers(i_s, i_d, cols, sem):
    ds = []
    for cc in range(3):
      ds.append(pltpu.async_copy(tabs[cc].at[i_s], cols[cc], sem))
      ds.append(pltpu.async_copy(tabs[cc].at[i_d], cols[3 + cc], sem))
    return ds

  def fire_writes(b, cols, sem):
    base = pl.multiple_of(wid * per_w + b * B1, B1)
    return [pltpu.async_copy(cols[r], g_hbm.at[r, pl.ds(base, B1)], sem)
            for r in range(6)]

  def pair(i, _):
    b0 = 2 * i
    ia = fire_idx(b0, isa, ida, semia)
    ib = fire_idx(b0 + 1, isb, idb, semib)
    for d in ia:
      d.wait()
    ga = fire_gathers(isa, ida, cols_a, semga)
    for d in ib:
      d.wait()
    gb = fire_gathers(isb, idb, cols_b, semgb)
    for d in ga:
      d.wait()
    wa = fire_writes(b0, cols_a, semwa)
    for d in gb:
      d.wait()
    wb = fire_writes(b0 + 1, cols_b, semwb)
    for d in wa + wb:
      d.wait()
    return _

  lax.fori_loop(0, nbatch // 2, pair, 0)
  if nbatch % 2:
    b0 = nbatch - 1
    ia = fire_idx(b0, isa, ida, semia)
    for d in ia:
      d.wait()
    ga = fire_gathers(isa, ida, cols_a, semga)
    for d in ga:
      d.wait()
    wa = fire_writes(b0, cols_a, semwa)
    for d in wa:
      d.wait()


def _sc_gather(posT, srcp, dstp, n_nodes):
  e_pad = srcp.shape[0]
  px, py, pz = posT[0], posT[1], posT[2]
  body = functools.partial(_sc_gather_body, n_nodes)
  fn = pl.kernel(
      body,
      out_type=jax.ShapeDtypeStruct((6, e_pad), jnp.float32),
      mesh=plsc.VectorSubcoreMesh(core_axis_name="c", subcore_axis_name="s"),
      scratch_types=[
          pltpu.MemorySpace.VMEM_SHARED((n_nodes,), jnp.float32),
          pltpu.MemorySpace.VMEM_SHARED((n_nodes,), jnp.float32),
          pltpu.MemorySpace.VMEM_SHARED((n_nodes,), jnp.float32),
          pltpu.VMEM((n_nodes // NS,), jnp.float32),
      ] + [pltpu.SemaphoreType.DMA] * 6
        + [pltpu.VMEM((B1,), jnp.int32)] * 4
        + [pltpu.VMEM((B1,), jnp.float32)] * 12,
  )
  return fn(px, py, pz, srcp, dstp)


def _tc_compute_body(e_real, br, kp_ref, g_ref, v_ref):
  pid = pl.program_id(0)
  fx, fy, fz = g_ref[0], g_ref[1], g_ref[2]
  rx = fx - g_ref[3]
  ry = fy - g_ref[4]
  rz = fz - g_ref[5]
  rows = lax.broadcasted_iota(jnp.int32, (br, 128), 0)
  lanes = lax.broadcasted_iota(jnp.int32, (br, 128), 1)
  gid = (pid * br + rows) * 128 + lanes
  valid = gid < e_real
  one = jnp.where(valid, 1.0, 0.0).astype(jnp.float32)
  fxm = fx * one
  fym = fy * one
  fzm = fz * one
  for k in range(K):
    dx = rx - kp_ref[k, 0]
    dy = ry - kp_ref[k, 1]
    dz = rz - kp_ref[k, 2]
    d2 = dx * dx + dy * dy + dz * dz + 1e-12
    dist = jnp.sqrt(d2)
    infl = jnp.maximum(0.0, 1.0 - dist * (1.0 / SIGMA))
    v_ref[3 * k + 0] = infl * fxm
    v_ref[3 * k + 1] = infl * fym
    v_ref[3 * k + 2] = infl * fzm
  v_ref[45] = one
  zero = jnp.zeros((br, 128), jnp.float32)
  v_ref[46] = zero
  v_ref[47] = zero


def _tc_compute(g3, kp, e_real):
  rows_tot = g3.shape[1]
  br = 256
  grid = rows_tot // br
  body = functools.partial(_tc_compute_body, e_real, br)
  return pl.pallas_call(
      body,
      grid=(grid,),
      in_specs=[
          pl.BlockSpec(memory_space=pltpu.SMEM),
          pl.BlockSpec((6, br, 128), lambda i: (0, i, 0)),
      ],
      out_specs=pl.BlockSpec((48, br, 128), lambda i: (0, i, 0)),
      out_shape=jax.ShapeDtypeStruct((48, rows_tot, 128), jnp.float32),
  )(kp, g3)


def _sc_scatter_body(n_nodes, e_pad, dst2_hbm, v4_hbm, acc_hbm,
                     acc_sh, zbuf, wbuf, semla, semlb, sems,
                     idxa, vbufa, idxb, vbufb):
  c = lax.axis_index("c")
  t = lax.axis_index("s")
  per_t = e_pad // NS
  nbatch = per_t // B3
  n_per_t = n_nodes // NS
  rows = B3 // 128

  def zgrp(g, _):
    zbuf[pl.ds(g * L, L)] = jnp.zeros((L,), jnp.float32)
    return _
  lax.fori_loop(0, n_per_t // L, zgrp, 0)

  def fire_loads(sg, b, idx, vbuf, sem):
    ebase = pl.multiple_of(t * per_t + b * B3, B3)
    rbase = pl.multiple_of(ebase // 128, rows)
    di = pltpu.async_copy(dst2_hbm.at[pl.ds(rbase, rows)], idx, sem)
    dv = pltpu.async_copy(
        v4_hbm.at[pl.ds(sg * 8, 8), pl.ds(rbase, rows), :], vbuf, sem)
    return di, dv

  def fire_scatters(idx, vbuf):
    ds = []
    for cl in range(8):
      col = acc_sh.at[pl.ds(pl.multiple_of(cl * n_nodes, 8), n_nodes)]
      for j in range(rows):
        ds.append(pltpu.async_copy(vbuf.at[cl, j], col.at[idx.at[j]],
                                   sems, add=True))
    return ds

  for sl in range(3):
    sg = 3 * c + sl
    for cl in range(8):
      off = pl.multiple_of(cl * n_nodes + t * n_per_t, n_per_t)
      pltpu.sync_copy(zbuf, acc_sh.at[pl.ds(off, n_per_t)])
    plsc.subcore_barrier()

    def batch2(i, _):
      b = 2 * i
      la = fire_loads(sg, b, idxa, vbufa, semla)
      lb = fire_loads(sg, b + 1, idxb, vbufb, semlb)
      for d in la:
        d.wait()
      sa = fire_scatters(idxa, vbufa)
      for d in lb:
        d.wait()
      sb = fire_scatters(idxb, vbufb)
      for d in sa + sb:
        d.wait()
      return _

    lax.fori_loop(0, nbatch // 2, batch2, 0)
    plsc.subcore_barrier()
    for cl in range(8):
      off = pl.multiple_of(cl * n_nodes + t * n_per_t, n_per_t)
      nbase = pl.multiple_of(t * n_per_t, n_per_t)
      pltpu.sync_copy(acc_sh.at[pl.ds(off, n_per_t)], wbuf)
      hoff = pl.multiple_of((sg * 8 + cl) * n_nodes + nbase, n_per_t)
      pltpu.sync_copy(wbuf, acc_hbm.at[pl.ds(hoff, n_per_t)])
    plsc.subcore_barrier()


def _sc_scatter(dst2, v4, n_nodes):
  e_pad = v4.shape[1] * 128
  body = functools.partial(_sc_scatter_body, n_nodes, e_pad)
  rows = B3 // 128
  fn = pl.kernel(
      body,
      out_type=jax.ShapeDtypeStruct((48 * n_nodes,), jnp.float32),
      mesh=plsc.VectorSubcoreMesh(core_axis_name="c", subcore_axis_name="s"),
      scratch_types=[
          pltpu.MemorySpace.VMEM_SHARED((8 * n_nodes,), jnp.float32),
          pltpu.VMEM((n_nodes // NS,), jnp.float32),
          pltpu.VMEM((n_nodes // NS,), jnp.float32),
          pltpu.SemaphoreType.DMA,
          pltpu.SemaphoreType.DMA,
          pltpu.SemaphoreType.DMA,
          pltpu.VMEM((rows, 128), jnp.int32),
          pltpu.VMEM((8, rows, 128), jnp.float32),
          pltpu.VMEM((rows, 128), jnp.int32),
          pltpu.VMEM((8, rows, 128), jnp.float32),
      ],
  )
  return fn(dst2, v4)


def _tc_out_body(bn, acc_ref, wf_ref, out_ref):
  acc2 = acc_ref[...].reshape(48, bn)
  mm = jax.lax.dot_general(acc2, wf_ref[...], (((0,), (0,)), ((), ())),
                           preferred_element_type=jnp.float32)
  deg = jax.lax.broadcast_in_dim(acc2[45], (bn, D_OUT), (0,))
  y = mm / jnp.maximum(deg, 1.0)
  out_ref[...] = jnp.where(y >= 0.0, y, 0.1 * y)


def _tc_out(acc3, wf):
  n_nodes = acc3.shape[2]
  bn = 5888
  grid = n_nodes // bn
  body = functools.partial(_tc_out_body, bn)
  return pl.pallas_call(
      body,
      grid=(grid,),
      in_specs=[
          pl.BlockSpec((6, 8, bn), lambda i: (0, 0, i)),
          pl.BlockSpec((48, D_OUT), lambda i: (0, 0)),
      ],
      out_specs=pl.BlockSpec((bn, D_OUT), lambda i: (i, 0)),
      out_shape=jax.ShapeDtypeStruct((n_nodes, D_OUT), jnp.float32),
  )(acc3, wf)


def kernel(points, edge_index, kernel_points, W):
  n_nodes = points.shape[0]
  e_real = edge_index.shape[1]
  step = NW * B1
  e_pad = ((e_real + step - 1) // step) * step

  n_pad = ((n_nodes + 127) // 128) * 128

  pos = points[:, 1:4].astype(jnp.float32)
  posT = jnp.pad(pos.T, ((0, 0), (0, n_pad - n_nodes)))
  src = edge_index[0].astype(jnp.int32)
  dst = edge_index[1].astype(jnp.int32)
  srcp = jnp.pad(src, (0, e_pad - e_real))
  dstp = jnp.pad(dst, (0, e_pad - e_real))

  kp = kernel_points.astype(jnp.float32)

  g = _sc_gather(posT, srcp, dstp, n_pad)
  g3 = g.reshape(6, e_pad // 128, 128)
  v3 = _tc_compute(g3, kp, e_real)

  dst2 = dstp.reshape(e_pad // 128, 128)
  acc = _sc_scatter(dst2, v3, n_pad).reshape(6, 8, n_pad)

  wf = W.reshape(K * 3, D_OUT).astype(jnp.float32)
  wf = jnp.concatenate([wf, jnp.zeros((3, D_OUT), jnp.float32)], axis=0)
  return _tc_out(acc, wf)[:n_nodes]
